# speedup vs baseline: 2.5716x; 2.5716x over previous
import functools

import jax
import jax.numpy as jnp
from jax import lax
from jax.experimental import pallas as pl
from jax.experimental.pallas import tpu as pltpu
from jax.experimental.pallas import tpu_sc as plsc

B = 4
R = 8192
D = 2048
NC, NS, L = 2, 16, 16
NW = NC * NS
ROWS = B * R
PER_W = ROWS // NW
WPB = R // PER_W
K = 16
NCH = PER_W // K


def _build_sc_gather():
    mesh = plsc.VectorSubcoreMesh(core_axis_name="c", subcore_axis_name="s")

    @functools.partial(
        pl.kernel,
        mesh=mesh,
        out_type=jax.ShapeDtypeStruct((ROWS, D), jnp.float32),
        scratch_types=[
            pltpu.VMEM((NCH, L), jnp.int32),
            pltpu.VMEM((K, D), jnp.float32),
            pltpu.SemaphoreType.DMA,
        ],
    )
    def body(x_hbm, perm_hbm, out_hbm, idx_v, buf, sem):
        wid = lax.axis_index("s") * NC + lax.axis_index("c")
        batch = wid // WPB
        part = wid % WPB
        pltpu.sync_copy(perm_hbm.at[part], idx_v)
        row_off = batch * R

        def add_off(j, _):
            idx_v[j, :] = idx_v[j, :] + row_off
            return 0

        lax.fori_loop(0, NCH, add_off, 0)

        out_base = wid * PER_W

        def chunk(k, _):
            pltpu.async_copy(x_hbm.at[idx_v.at[k]], buf, sem).wait()
            pltpu.sync_copy(buf, out_hbm.at[pl.ds(out_base + k * K, K)])
            return 0

        lax.fori_loop(0, NCH, chunk, 0)

    return body


_sc_gather = _build_sc_gather()


def kernel(inputs, perm):
    x = inputs.reshape(ROWS, D)
    perm3 = perm.reshape(WPB, NCH, L)
    out = _sc_gather(x, perm3)
    return out.reshape(B, R, D)


# speedup vs baseline: 3.1082x; 1.2087x over previous
import functools

import jax
import jax.numpy as jnp
from jax import lax
from jax.experimental import pallas as pl
from jax.experimental.pallas import tpu as pltpu
from jax.experimental.pallas import tpu_sc as plsc

B = 4
R = 8192
D = 2048
NC, NS, L = 2, 16, 16
NW = NC * NS
ROWS = B * R
PER_W = ROWS // NW
WPB = R // PER_W
K = 16
NCH = PER_W // K


def _build_sc_gather():
    mesh = plsc.VectorSubcoreMesh(core_axis_name="c", subcore_axis_name="s")

    @functools.partial(
        pl.kernel,
        mesh=mesh,
        out_type=jax.ShapeDtypeStruct((ROWS, D), jnp.float32),
        scratch_types=[
            pltpu.VMEM((NCH, L), jnp.int32),
            pltpu.VMEM((K, D), jnp.float32),
            pltpu.VMEM((K, D), jnp.float32),
            pltpu.SemaphoreType.DMA,
            pltpu.SemaphoreType.DMA,
        ],
    )
    def body(x_hbm, perm_hbm, out_hbm, idx_v, buf0, buf1, sem0, sem1):
        wid = lax.axis_index("s") * NC + lax.axis_index("c")
        batch = wid // WPB
        part = wid % WPB
        pltpu.sync_copy(perm_hbm.at[part], idx_v)
        row_off = batch * R

        def add_off(j, _):
            idx_v[j, :] = idx_v[j, :] + row_off
            return 0

        lax.fori_loop(0, NCH, add_off, 0)

        out_base = wid * PER_W

        pltpu.async_copy(x_hbm.at[idx_v.at[0]], buf0, sem0)
        NP = NCH // 2

        def pair(p, _):
            k0 = p * 2
            pltpu.async_copy(x_hbm.at[idx_v.at[k0 + 1]], buf1, sem1)
            pltpu.make_async_copy(x_hbm.at[idx_v.at[k0]], buf0, sem0).wait()
            pltpu.sync_copy(buf0, out_hbm.at[pl.ds(out_base + k0 * K, K)])

            @pl.when(p + 1 < NP)
            def _():
                pltpu.async_copy(x_hbm.at[idx_v.at[k0 + 2]], buf0, sem0)

            pltpu.make_async_copy(x_hbm.at[idx_v.at[k0 + 1]], buf1, sem1).wait()
            pltpu.sync_copy(buf1, out_hbm.at[pl.ds(out_base + (k0 + 1) * K, K)])
            return 0

        lax.fori_loop(0, NP, pair, 0)

    return body


_sc_gather = _build_sc_gather()


def kernel(inputs, perm):
    x = inputs.reshape(ROWS, D)
    perm3 = perm.reshape(WPB, NCH, L)
    out = _sc_gather(x, perm3)
    return out.reshape(B, R, D)
